# Initial kernel scaffold; baseline (speedup 1.0000x reference)
import jax
import jax.numpy as jnp
from jax.experimental import pallas as pl


def kernel(x, edge_index, W1_int, b1_int, W1_nh, b1_nh, W2_int, b2_int, W2_nh, b2_nh, w_att, b_att, W_d, b_d):
    raise NotImplementedError("write your pallas kernel here")


# speedup vs baseline: 26.8532x; 26.8532x over previous
import functools

import jax
import jax.numpy as jnp
from jax import lax
from jax.experimental import pallas as pl
from jax.experimental.pallas import tpu as pltpu
from jax.experimental.pallas import tpu_sc as plsc

_N = 100000
_E = 6400000
_F = 16
_EC = 128
_U = _E // (8 * _EC)
_NW = 32
_WR = 6248
_WREM = _N - 16 * _WR


def _unit(h_hbm, sidx, didx, rows, acc, gsem, ssem, a):
    gds = [pltpu.async_copy(h_hbm.at[sidx.at[a, b]], rows.at[a * 8 + b], gsem)
           for b in range(8)]
    for d in gds:
        d.wait()
    sds = [pltpu.async_copy(rows.at[a * 8 + b], acc.at[didx.at[a, b]],
                            ssem, add=True) for b in range(8)]
    for d in sds:
        d.wait()


def _edge_range(h_hbm, src_hbm, dst_hbm, sidx, didx, rows, acc, gsem, ssem,
                q0, nsup):
    @pl.loop(0, nsup)
    def _super(t):
        q = q0 + t * 2
        pltpu.sync_copy(src_hbm.at[pl.ds(q, 2)], sidx)
        pltpu.sync_copy(dst_hbm.at[pl.ds(q, 2)], didx)
        _unit(h_hbm, sidx, didx, rows, acc, gsem, ssem, 0)
        _unit(h_hbm, sidx, didx, rows, acc, gsem, ssem, 1)


def _one_unit(h_hbm, src_hbm, dst_hbm, sidx, didx, rows, acc, gsem, ssem, q):
    pltpu.sync_copy(src_hbm.at[pl.ds(q, 1)], sidx.at[pl.ds(0, 1)])
    pltpu.sync_copy(dst_hbm.at[pl.ds(q, 1)], didx.at[pl.ds(0, 1)])
    _unit(h_hbm, sidx, didx, rows, acc, gsem, ssem, 0)


def _zero_acc(zini_hbm, acc, s):
    pltpu.sync_copy(zini_hbm, acc.at[pl.ds(s * _WR, _WR)])

    @pl.when(s == 15)
    def _zrem():
        pltpu.sync_copy(zini_hbm.at[pl.ds(0, _WREM)],
                        acc.at[pl.ds(16 * _WR, _WREM)])


def _write_out(acc, out_hbm, c, s):
    pltpu.sync_copy(acc.at[pl.ds(s * _WR, _WR)],
                    out_hbm.at[c, pl.ds(s * _WR, _WR)])

    @pl.when(s == 15)
    def _wrem():
        pltpu.sync_copy(acc.at[pl.ds(16 * _WR, _WREM)],
                        out_hbm.at[c, pl.ds(16 * _WR, _WREM)])


_UPS = _U // 16
_LU2 = _U - 16 * _UPS


def _sc_scatter_body(ha_hbm, hb_hbm, src_hbm, dst_hbm, zini_hbm, out_hbm,
                      sidx, didx, rows, acc, gsem, ssem):
    c = lax.axis_index("c")
    s = lax.axis_index("s")
    _zero_acc(zini_hbm, acc, s)
    plsc.subcore_barrier()

    q0 = s * _UPS

    @pl.when(c == 0)
    def _feat_lo():
        _edge_range(ha_hbm, src_hbm, dst_hbm, sidx, didx, rows, acc,
                    gsem, ssem, q0, _UPS // 2)

        @pl.when(s < _LU2)
        def _extra():
            _one_unit(ha_hbm, src_hbm, dst_hbm, sidx, didx, rows, acc,
                      gsem, ssem, 16 * _UPS + s)

    @pl.when(c == 1)
    def _feat_hi():
        _edge_range(hb_hbm, src_hbm, dst_hbm, sidx, didx, rows, acc,
                    gsem, ssem, q0, _UPS // 2)

        @pl.when(s < _LU2)
        def _extra():
            _one_unit(hb_hbm, src_hbm, dst_hbm, sidx, didx, rows, acc,
                      gsem, ssem, 16 * _UPS + s)

    plsc.subcore_barrier()
    _write_out(acc, out_hbm, c, s)


_sc_scatter = functools.partial(
    pl.kernel,
    out_type=jax.ShapeDtypeStruct((2, _N, 8), jnp.float32),
    mesh=plsc.VectorSubcoreMesh(core_axis_name="c", subcore_axis_name="s"),
    compiler_params=pltpu.CompilerParams(use_tc_tiling_on_sc=False),
    scratch_types=[
        pltpu.VMEM((2, 8, _EC), jnp.int32),
        pltpu.VMEM((2, 8, _EC), jnp.int32),
        pltpu.VMEM((16, _EC, 8), jnp.float32),
        pltpu.VMEM_SHARED((_N, 8), jnp.float32),
        pltpu.SemaphoreType.DMA,
        pltpu.SemaphoreType.DMA,
    ],
)(_sc_scatter_body)


_R = 4000


def _dense1_body(x_ref, a_ref, wi_ref, bi_ref, wn_ref, bn_ref,
                 h1_ref, ha_ref, hb_ref):
    xb = x_ref[...]
    ab = jnp.concatenate([a_ref[0], a_ref[1]], axis=1)
    hi = jnp.dot(xb, wi_ref[...], preferred_element_type=jnp.float32)
    hn = jnp.dot(ab, wn_ref[...], preferred_element_type=jnp.float32)
    h1 = (jnp.maximum(hi + bi_ref[...], 0.0)
          + jnp.maximum(hn + bn_ref[...], 0.0))
    h1_ref[...] = h1
    ha_ref[...] = h1[:, :8]
    hb_ref[...] = h1[:, 8:]


def _dense1(x16, agg, wi, bi, wn, bn):
    g = _N // _R
    return pl.pallas_call(
        _dense1_body,
        grid=(g,),
        in_specs=[
            pl.BlockSpec((_R, _F), lambda i: (i, 0)),
            pl.BlockSpec((2, _R, 8), lambda i: (0, i, 0)),
            pl.BlockSpec((_F, _F), lambda i: (0, 0)),
            pl.BlockSpec((1, _F), lambda i: (0, 0)),
            pl.BlockSpec((_F, _F), lambda i: (0, 0)),
            pl.BlockSpec((1, _F), lambda i: (0, 0)),
        ],
        out_specs=[
            pl.BlockSpec((_R, _F), lambda i: (i, 0)),
            pl.BlockSpec((_R, 8), lambda i: (i, 0)),
            pl.BlockSpec((_R, 8), lambda i: (i, 0)),
        ],
        out_shape=[
            jax.ShapeDtypeStruct((_N, _F), jnp.float32),
            jax.ShapeDtypeStruct((_N, 8), jnp.float32),
            jax.ShapeDtypeStruct((_N, 8), jnp.float32),
        ],
    )(x16, agg, wi, bi, wn, bn)


def _dense2_body(h1_ref, a_ref, wi_ref, bi_ref, wn_ref, bn_ref,
                 watt_ref, batt_ref, wd_ref, e_ref, u_ref, ssum_ref):
    i = pl.program_id(0)
    hb = h1_ref[...]
    ab = jnp.concatenate([a_ref[0], a_ref[1]], axis=1)
    zi = jnp.maximum(
        jnp.dot(hb, wi_ref[...], preferred_element_type=jnp.float32)
        + bi_ref[...], 0.0)
    zn = jnp.maximum(
        jnp.dot(ab, wn_ref[...], preferred_element_type=jnp.float32)
        + bn_ref[...], 0.0)
    z = jnp.concatenate([zi, zn], axis=1)
    t = jnp.tanh(
        jnp.dot(z, watt_ref[...], preferred_element_type=jnp.float32)
        + batt_ref[0, 0])
    e = jnp.exp(t)
    e_ref[...] = e
    u_ref[...] = jnp.dot(z, wd_ref[...], preferred_element_type=jnp.float32)

    @pl.when(i == 0)
    def _init():
        ssum_ref[0, 0] = 0.0

    ssum_ref[0, 0] += jnp.sum(e)


def _dense2(h1, agg, wi, bi, wn, bn, watt, batt, wd):
    g = _N // _R
    return pl.pallas_call(
        _dense2_body,
        grid=(g,),
        in_specs=[
            pl.BlockSpec((_R, _F), lambda i: (i, 0)),
            pl.BlockSpec((2, _R, 8), lambda i: (0, i, 0)),
            pl.BlockSpec((_F, 32), lambda i: (0, 0)),
            pl.BlockSpec((1, 32), lambda i: (0, 0)),
            pl.BlockSpec((_F, 32), lambda i: (0, 0)),
            pl.BlockSpec((1, 32), lambda i: (0, 0)),
            pl.BlockSpec((64, 1), lambda i: (0, 0)),
            pl.BlockSpec(memory_space=pltpu.SMEM),
            pl.BlockSpec((64, 1), lambda i: (0, 0)),
        ],
        out_specs=[
            pl.BlockSpec((_R, 1), lambda i: (i, 0)),
            pl.BlockSpec((_R, 1), lambda i: (i, 0)),
            pl.BlockSpec(memory_space=pltpu.SMEM),
        ],
        out_shape=[
            jax.ShapeDtypeStruct((_N, 1), jnp.float32),
            jax.ShapeDtypeStruct((_N, 1), jnp.float32),
            jax.ShapeDtypeStruct((1, 1), jnp.float32),
        ],
    )(h1, agg, wi, bi, wn, bn, watt, batt, wd)


def _final_body(e_ref, u_ref, ssum_ref, bd_ref, o_ref):
    o_ref[...] = (e_ref[...] * u_ref[...] * (1.0 / ssum_ref[0, 0])
                  + bd_ref[0, 0])


def _final(e, u, ssum, bd):
    g = _N // _R
    return pl.pallas_call(
        _final_body,
        grid=(g,),
        in_specs=[
            pl.BlockSpec((_R, 1), lambda i: (i, 0)),
            pl.BlockSpec((_R, 1), lambda i: (i, 0)),
            pl.BlockSpec(memory_space=pltpu.SMEM),
            pl.BlockSpec(memory_space=pltpu.SMEM),
        ],
        out_specs=pl.BlockSpec((_R, 1), lambda i: (i, 0)),
        out_shape=jax.ShapeDtypeStruct((_N, 1), jnp.float32),
    )(e, u, ssum, bd)


def kernel(x, edge_index, W1_int, b1_int, W1_nh, b1_nh,
           W2_int, b2_int, W2_nh, b2_nh, w_att, b_att, W_d, b_d):
    x16 = jnp.pad(x, ((0, 0), (0, _F - 11)))
    xa = x16[:, :8]
    xb = x16[:, 8:]
    src2d = edge_index[0].reshape(_U, 8, _EC)
    dst2d = edge_index[1].reshape(_U, 8, _EC)
    zini = jnp.zeros((_WR, 8), jnp.float32)
    w1i = jnp.pad(W1_int, ((0, _F - 11), (0, 0)))
    w1n = jnp.pad(W1_nh, ((0, _F - 11), (0, 0)))

    agg1 = _sc_scatter(xa, xb, src2d, dst2d, zini)
    h1, h1a, h1b = _dense1(x16, agg1, w1i, b1_int.reshape(1, _F),
                           w1n, b1_nh.reshape(1, _F))
    agg2 = _sc_scatter(h1a, h1b, src2d, dst2d, zini)
    e, u, ssum = _dense2(h1, agg2, W2_int, b2_int.reshape(1, 32),
                         W2_nh, b2_nh.reshape(1, 32),
                         w_att.reshape(64, 1), b_att.reshape(1, 1), W_d)
    out = _final(e, u, ssum, b_d.reshape(1, 1))
    return out[:, 0]


# speedup vs baseline: 36.9237x; 1.3750x over previous
import functools

import jax
import jax.numpy as jnp
from jax import lax
from jax.experimental import pallas as pl
from jax.experimental.pallas import tpu as pltpu
from jax.experimental.pallas import tpu_sc as plsc

_N = 100000
_E = 6400000
_F = 16
_EC = 128
_U = _E // (8 * _EC)
_NW = 32
_WR = 6248
_WREM = _N - 16 * _WR


_SB = 13


def _fire_gathers(h_hbm, idx, rows, gsem, uu, p):
    for b in range(8):
        pltpu.async_copy(h_hbm.at[idx.at[uu, b]], rows.at[p, b], gsem)


def _fire_scatters(acc, idx, rows, ssem, uu, p):
    for b in range(8):
        pltpu.async_copy(rows.at[p, b], acc.at[idx.at[uu, b]], ssem, add=True)


def _drain(rows, sem, hbm_dummy, n):
    for _ in range(n):
        for b in range(8):
            pltpu.make_async_copy(hbm_dummy.at[pl.ds(0, _EC)],
                                  rows.at[0, b], sem).wait()


def _edge_range(h_hbm, src_hbm, dst_hbm, sidx, didx, rows, acc, gsem, ssem,
                q0, nsb):
    @pl.loop(0, nsb)
    def _sb(t):
        q = q0 + t * _SB
        pltpu.sync_copy(src_hbm.at[pl.ds(q, _SB)], sidx)
        pltpu.sync_copy(dst_hbm.at[pl.ds(q, _SB)], didx)
        _fire_gathers(h_hbm, sidx, rows, gsem, 0, 0)

        @pl.loop(0, _SB)
        def _u(u):
            p = lax.rem(u, 2)

            @pl.when(u > 0)
            def _ws():
                _drain(rows, ssem, h_hbm, 1)

            @pl.when(u < _SB - 1)
            def _fg():
                _fire_gathers(h_hbm, sidx, rows, gsem, u + 1, 1 - p)

            _drain(rows, gsem, h_hbm, 1)
            _fire_scatters(acc, didx, rows, ssem, u, p)

        _drain(rows, ssem, h_hbm, 1)


def _one_unit(h_hbm, src_hbm, dst_hbm, sidx, didx, rows, acc, gsem, ssem, q):
    pltpu.sync_copy(src_hbm.at[pl.ds(q, 1)], sidx.at[pl.ds(0, 1)])
    pltpu.sync_copy(dst_hbm.at[pl.ds(q, 1)], didx.at[pl.ds(0, 1)])
    _fire_gathers(h_hbm, sidx, rows, gsem, 0, 0)
    _drain(rows, gsem, h_hbm, 1)
    _fire_scatters(acc, didx, rows, ssem, 0, 0)
    _drain(rows, ssem, h_hbm, 1)


def _zero_acc(zini_hbm, acc, s):
    pltpu.sync_copy(zini_hbm, acc.at[pl.ds(s * _WR, _WR)])

    @pl.when(s == 15)
    def _zrem():
        pltpu.sync_copy(zini_hbm.at[pl.ds(0, _WREM)],
                        acc.at[pl.ds(16 * _WR, _WREM)])


def _write_out(acc, out_hbm, c, s):
    pltpu.sync_copy(acc.at[pl.ds(s * _WR, _WR)],
                    out_hbm.at[c, pl.ds(s * _WR, _WR)])

    @pl.when(s == 15)
    def _wrem():
        pltpu.sync_copy(acc.at[pl.ds(16 * _WR, _WREM)],
                        out_hbm.at[c, pl.ds(16 * _WR, _WREM)])


_UPS = _U // 16
_LU2 = _U - 16 * _UPS


def _sc_scatter_body(ha_hbm, hb_hbm, src_hbm, dst_hbm, zini_hbm, out_hbm,
                      sidx, didx, rows, acc, gsem, ssem):
    c = lax.axis_index("c")
    s = lax.axis_index("s")
    _zero_acc(zini_hbm, acc, s)
    plsc.subcore_barrier()

    q0 = s * _UPS

    @pl.when(c == 0)
    def _feat_lo():
        _edge_range(ha_hbm, src_hbm, dst_hbm, sidx, didx, rows, acc,
                    gsem, ssem, q0, _UPS // _SB)

        @pl.when(s < _LU2)
        def _extra():
            _one_unit(ha_hbm, src_hbm, dst_hbm, sidx, didx, rows, acc,
                      gsem, ssem, 16 * _UPS + s)

    @pl.when(c == 1)
    def _feat_hi():
        _edge_range(hb_hbm, src_hbm, dst_hbm, sidx, didx, rows, acc,
                    gsem, ssem, q0, _UPS // _SB)

        @pl.when(s < _LU2)
        def _extra():
            _one_unit(hb_hbm, src_hbm, dst_hbm, sidx, didx, rows, acc,
                      gsem, ssem, 16 * _UPS + s)

    plsc.subcore_barrier()
    _write_out(acc, out_hbm, c, s)


_sc_scatter = functools.partial(
    pl.kernel,
    out_type=jax.ShapeDtypeStruct((2, _N, 8), jnp.float32),
    mesh=plsc.VectorSubcoreMesh(core_axis_name="c", subcore_axis_name="s"),
    compiler_params=pltpu.CompilerParams(use_tc_tiling_on_sc=False),
    scratch_types=[
        pltpu.VMEM((_SB, 8, _EC), jnp.int32),
        pltpu.VMEM((_SB, 8, _EC), jnp.int32),
        pltpu.VMEM((2, 8, _EC, 8), jnp.float32),
        pltpu.VMEM_SHARED((_N, 8), jnp.float32),
        pltpu.SemaphoreType.DMA,
        pltpu.SemaphoreType.DMA,
    ],
)(_sc_scatter_body)


_R = 4000


def _dense1_body(x_ref, a_ref, wi_ref, bi_ref, wn_ref, bn_ref,
                 h1_ref, ha_ref, hb_ref):
    xb = x_ref[...]
    ab = jnp.concatenate([a_ref[0], a_ref[1]], axis=1)
    hi = jnp.dot(xb, wi_ref[...], preferred_element_type=jnp.float32)
    hn = jnp.dot(ab, wn_ref[...], preferred_element_type=jnp.float32)
    h1 = (jnp.maximum(hi + bi_ref[...], 0.0)
          + jnp.maximum(hn + bn_ref[...], 0.0))
    h1_ref[...] = h1
    ha_ref[...] = h1[:, :8]
    hb_ref[...] = h1[:, 8:]


def _dense1(x16, agg, wi, bi, wn, bn):
    g = _N // _R
    return pl.pallas_call(
        _dense1_body,
        grid=(g,),
        in_specs=[
            pl.BlockSpec((_R, _F), lambda i: (i, 0)),
            pl.BlockSpec((2, _R, 8), lambda i: (0, i, 0)),
            pl.BlockSpec((_F, _F), lambda i: (0, 0)),
            pl.BlockSpec((1, _F), lambda i: (0, 0)),
            pl.BlockSpec((_F, _F), lambda i: (0, 0)),
            pl.BlockSpec((1, _F), lambda i: (0, 0)),
        ],
        out_specs=[
            pl.BlockSpec((_R, _F), lambda i: (i, 0)),
            pl.BlockSpec((_R, 8), lambda i: (i, 0)),
            pl.BlockSpec((_R, 8), lambda i: (i, 0)),
        ],
        out_shape=[
            jax.ShapeDtypeStruct((_N, _F), jnp.float32),
            jax.ShapeDtypeStruct((_N, 8), jnp.float32),
            jax.ShapeDtypeStruct((_N, 8), jnp.float32),
        ],
    )(x16, agg, wi, bi, wn, bn)


def _dense2_body(h1_ref, a_ref, wi_ref, bi_ref, wn_ref, bn_ref,
                 watt_ref, batt_ref, wd_ref, e_ref, u_ref, ssum_ref):
    i = pl.program_id(0)
    hb = h1_ref[...]
    ab = jnp.concatenate([a_ref[0], a_ref[1]], axis=1)
    zi = jnp.maximum(
        jnp.dot(hb, wi_ref[...], preferred_element_type=jnp.float32)
        + bi_ref[...], 0.0)
    zn = jnp.maximum(
        jnp.dot(ab, wn_ref[...], preferred_element_type=jnp.float32)
        + bn_ref[...], 0.0)
    z = jnp.concatenate([zi, zn], axis=1)
    t = jnp.tanh(
        jnp.dot(z, watt_ref[...], preferred_element_type=jnp.float32)
        + batt_ref[0, 0])
    e = jnp.exp(t)
    e_ref[...] = e
    u_ref[...] = jnp.dot(z, wd_ref[...], preferred_element_type=jnp.float32)

    @pl.when(i == 0)
    def _init():
        ssum_ref[0, 0] = 0.0

    ssum_ref[0, 0] += jnp.sum(e)


def _dense2(h1, agg, wi, bi, wn, bn, watt, batt, wd):
    g = _N // _R
    return pl.pallas_call(
        _dense2_body,
        grid=(g,),
        in_specs=[
            pl.BlockSpec((_R, _F), lambda i: (i, 0)),
            pl.BlockSpec((2, _R, 8), lambda i: (0, i, 0)),
            pl.BlockSpec((_F, 32), lambda i: (0, 0)),
            pl.BlockSpec((1, 32), lambda i: (0, 0)),
            pl.BlockSpec((_F, 32), lambda i: (0, 0)),
            pl.BlockSpec((1, 32), lambda i: (0, 0)),
            pl.BlockSpec((64, 1), lambda i: (0, 0)),
            pl.BlockSpec(memory_space=pltpu.SMEM),
            pl.BlockSpec((64, 1), lambda i: (0, 0)),
        ],
        out_specs=[
            pl.BlockSpec((_R, 1), lambda i: (i, 0)),
            pl.BlockSpec((_R, 1), lambda i: (i, 0)),
            pl.BlockSpec(memory_space=pltpu.SMEM),
        ],
        out_shape=[
            jax.ShapeDtypeStruct((_N, 1), jnp.float32),
            jax.ShapeDtypeStruct((_N, 1), jnp.float32),
            jax.ShapeDtypeStruct((1, 1), jnp.float32),
        ],
    )(h1, agg, wi, bi, wn, bn, watt, batt, wd)


def _final_body(e_ref, u_ref, ssum_ref, bd_ref, o_ref):
    o_ref[...] = (e_ref[...] * u_ref[...] * (1.0 / ssum_ref[0, 0])
                  + bd_ref[0, 0])


def _final(e, u, ssum, bd):
    g = _N // _R
    return pl.pallas_call(
        _final_body,
        grid=(g,),
        in_specs=[
            pl.BlockSpec((_R, 1), lambda i: (i, 0)),
            pl.BlockSpec((_R, 1), lambda i: (i, 0)),
            pl.BlockSpec(memory_space=pltpu.SMEM),
            pl.BlockSpec(memory_space=pltpu.SMEM),
        ],
        out_specs=pl.BlockSpec((_R, 1), lambda i: (i, 0)),
        out_shape=jax.ShapeDtypeStruct((_N, 1), jnp.float32),
    )(e, u, ssum, bd)


def kernel(x, edge_index, W1_int, b1_int, W1_nh, b1_nh,
           W2_int, b2_int, W2_nh, b2_nh, w_att, b_att, W_d, b_d):
    x16 = jnp.pad(x, ((0, 0), (0, _F - 11)))
    xa = x16[:, :8]
    xb = x16[:, 8:]
    src2d = edge_index[0].reshape(_U, 8, _EC)
    dst2d = edge_index[1].reshape(_U, 8, _EC)
    zini = jnp.zeros((_WR, 8), jnp.float32)
    w1i = jnp.pad(W1_int, ((0, _F - 11), (0, 0)))
    w1n = jnp.pad(W1_nh, ((0, _F - 11), (0, 0)))

    agg1 = _sc_scatter(xa, xb, src2d, dst2d, zini)
    h1, h1a, h1b = _dense1(x16, agg1, w1i, b1_int.reshape(1, _F),
                           w1n, b1_nh.reshape(1, _F))
    agg2 = _sc_scatter(h1a, h1b, src2d, dst2d, zini)
    e, u, ssum = _dense2(h1, agg2, W2_int, b2_int.reshape(1, 32),
                         W2_nh, b2_nh.reshape(1, 32),
                         w_att.reshape(64, 1), b_att.reshape(1, 1), W_d)
    out = _final(e, u, ssum, b_d.reshape(1, 1))
    return out[:, 0]
